# baseline (speedup 1.0000x reference)
import functools

import jax
import jax.numpy as jnp
from jax import lax
from jax.experimental import pallas as pl
from jax.experimental.pallas import tpu as pltpu
from jax.experimental.pallas import tpu_sc as plsc

NC = 2
NS = 16
NW = NC * NS
CW = 80
NPAD = 10240
SR = 80


def _sc_mesh():
    return plsc.VectorSubcoreMesh(core_axis_name="c", subcore_axis_name="s")


def _sc_gather_staged(planes, idx3):
    p_, nrows, hd = planes.shape
    nw, ch, cw = idx3.shape
    per_w = ch * cw
    rps = NPAD // NS
    nck = rps // SR

    @functools.partial(
        pl.kernel,
        out_type=jax.ShapeDtypeStruct((nw * per_w, hd), jnp.float32),
        mesh=_sc_mesh(),
        scratch_types=[
            pltpu.VMEM((ch, cw), jnp.int32),
            pltpu.VMEM((cw, hd), jnp.float32),
            pltpu.VMEM((cw, hd), jnp.float32),
            pltpu.VMEM((SR, hd), jnp.float32),
            pltpu.VMEM((SR, hd), jnp.float32),
            pltpu.VMEM_SHARED((NPAD, hd), jnp.float32),
            pltpu.SemaphoreType.DMA,
            pltpu.SemaphoreType.DMA,
            pltpu.SemaphoreType.DMA,
            pltpu.SemaphoreType.DMA,
        ],
    )
    def k(pln, idx, out, idx_v, b0, b1, sa, sb, acc, sg0, sg1, sw0, sw1):
        cid = lax.axis_index("c")
        sid = lax.axis_index("s")
        wid = sid * NC + cid
        base = wid * per_w
        pltpu.sync_copy(idx.at[wid], idx_v)

        def stage_body(k2, carry):
            r0 = sid * rps + k2 * SR
            pltpu.sync_copy(pln.at[0, pl.ds(r0, SR)], sa)
            if p_ == 2:
                pltpu.sync_copy(pln.at[1, pl.ds(r0, SR)], sb)

                def add_body(i, c2):
                    rr = i // 8
                    cc = (i % 8) * 16
                    sa[rr, pl.ds(cc, 16)] = (sa[rr, pl.ds(cc, 16)]
                                             + sb[rr, pl.ds(cc, 16)])
                    return c2

                lax.fori_loop(0, SR * 8, add_body, 0)
            pltpu.sync_copy(sa, acc.at[pl.ds(r0, SR)])
            return carry

        nck_i = jnp.minimum(nck, jnp.maximum(0, (nrows - sid * rps) // SR))
        lax.fori_loop(0, nck_i, stage_body, 0)
        plsc.subcore_barrier()

        pltpu.async_copy(acc.at[idx_v.at[0]], b0, sg0)

        def body(kk, carry):
            j0 = 2 * kk
            c1 = pltpu.async_copy(acc.at[idx_v.at[j0 + 1]], b1, sg1)
            pltpu.make_async_copy(acc.at[idx_v.at[j0]], b0, sg0).wait()
            w0 = pltpu.async_copy(b0, out.at[pl.ds(base + j0 * cw, cw)], sw0)
            c1.wait()
            w1 = pltpu.async_copy(b1, out.at[pl.ds(base + (j0 + 1) * cw, cw)], sw1)
            w0.wait()

            @pl.when(j0 + 2 < ch)
            def _():
                pltpu.async_copy(acc.at[idx_v.at[j0 + 2]], b0, sg0)

            w1.wait()
            return carry

        lax.fori_loop(0, ch // 2, body, 0)
        if ch % 2 == 1:
            jl = ch - 1
            pltpu.make_async_copy(acc.at[idx_v.at[jl]], b0, sg0).wait()
            pltpu.sync_copy(b0, out.at[pl.ds(base + jl * cw, cw)])

    return k(planes, idx3)


def _sc_scatter(hmat, idx3, init):
    nw, ch, cw = idx3.shape
    hd = hmat.shape[1]
    per_w = ch * cw
    rps = NPAD // NS
    chained = init.ndim == 3

    @functools.partial(
        pl.kernel,
        out_type=jax.ShapeDtypeStruct((NC, NPAD, hd), jnp.float32),
        mesh=_sc_mesh(),
        scratch_types=[
            pltpu.VMEM((ch, cw), jnp.int32),
            pltpu.VMEM((cw, hd), jnp.float32),
            pltpu.VMEM((cw, hd), jnp.float32),
            pltpu.VMEM_SHARED((NPAD, hd), jnp.float32),
            pltpu.SemaphoreType.DMA,
            pltpu.SemaphoreType.DMA,
            pltpu.SemaphoreType.DMA,
            pltpu.SemaphoreType.DMA,
        ],
    )
    def k(h, idx, zb, out, idx_v, b0, b1, acc, sl0, sl1, ss0, ss1):
        cid = lax.axis_index("c")
        sid = lax.axis_index("s")
        wid = sid * NC + cid
        base = wid * per_w
        pltpu.sync_copy(idx.at[wid], idx_v)
        if chained:
            pltpu.sync_copy(zb.at[cid, pl.ds(sid * rps, rps)],
                            acc.at[pl.ds(sid * rps, rps)])
        else:
            pltpu.sync_copy(zb, acc.at[pl.ds(sid * rps, rps)])
        plsc.subcore_barrier()
        pltpu.async_copy(h.at[pl.ds(base, cw)], b0, sl0)

        def body(kk, carry):
            j0 = 2 * kk
            c1 = pltpu.async_copy(h.at[pl.ds(base + (j0 + 1) * cw, cw)], b1, sl1)
            pltpu.make_async_copy(h.at[pl.ds(base, cw)], b0, sl0).wait()
            s0 = pltpu.async_copy(b0, acc.at[idx_v.at[j0]], ss0, add=True)
            c1.wait()
            s1 = pltpu.async_copy(b1, acc.at[idx_v.at[j0 + 1]], ss1, add=True)
            s0.wait()

            @pl.when(j0 + 2 < ch)
            def _():
                pltpu.async_copy(h.at[pl.ds(base + (j0 + 2) * cw, cw)], b0, sl0)

            s1.wait()

            @pl.when(j0 + 3 < ch)
            def _():
                pltpu.async_copy(h.at[pl.ds(base + (j0 + 3) * cw, cw)], b1, sl1)

            return carry

        lax.fori_loop(0, ch // 2, body, 0)
        if ch % 2 == 1:
            jl = ch - 1
            pltpu.make_async_copy(h.at[pl.ds(base, cw)], b0, sl0).wait()
            pltpu.sync_copy(b0, acc.at[idx_v.at[jl]], add=True)
        plsc.subcore_barrier()
        pltpu.sync_copy(acc.at[pl.ds(sid * rps, rps)],
                        out.at[cid, pl.ds(sid * rps, rps)])

    return k(hmat, idx3, init)


def _tc_h0(g0, eat, w1, w2, b2):
    e_, hd = g0.shape[0], w1.shape[1]
    r = 2560
    fe = eat.shape[0]

    def body(g_ref, e_ref, w1_ref, w2_ref, b_ref, o_ref):
        acc = jnp.dot(g_ref[...], w1_ref[...], preferred_element_type=jnp.float32)
        acc = acc + lax.dot_general(
            e_ref[...], w2_ref[...], (((0,), (0,)), ((), ())),
            preferred_element_type=jnp.float32)
        o_ref[...] = jnp.maximum(acc + b_ref[...], 0.0)

    return pl.pallas_call(
        body,
        grid=(e_ // r,),
        in_specs=[
            pl.BlockSpec((r, g0.shape[1]), lambda i: (i, 0)),
            pl.BlockSpec((fe, r), lambda i: (0, i)),
            pl.BlockSpec(w1.shape, lambda i: (0, 0)),
            pl.BlockSpec(w2.shape, lambda i: (0, 0)),
            pl.BlockSpec((1, hd), lambda i: (0, 0)),
        ],
        out_specs=pl.BlockSpec((r, hd), lambda i: (i, 0)),
        out_shape=jax.ShapeDtypeStruct((e_, hd), jnp.float32),
        compiler_params=pltpu.CompilerParams(dimension_semantics=("parallel",)),
    )(g0, eat, w1, w2, b2)


def _tc_conv(gm, h, h0, wc, bc2):
    e_, hd = h.shape
    r = 2560

    def body(g_ref, h_ref, h0_ref, w_ref, b_ref, o_ref):
        hb = h_ref[...]
        up = jnp.concatenate([hb[1:], hb[:1]], axis=0)
        dn = jnp.concatenate([hb[-1:], hb[:-1]], axis=0)
        even = (lax.broadcasted_iota(jnp.int32, (r, 1), 0) % 2) == 0
        rev = jnp.where(even, up, dn)
        d = g_ref[...] - rev
        acc = jnp.dot(d, w_ref[...], preferred_element_type=jnp.float32)
        o_ref[...] = jnp.maximum(acc + b_ref[...] + h0_ref[...], 0.0)

    return pl.pallas_call(
        body,
        grid=(e_ // r,),
        in_specs=[
            pl.BlockSpec((r, hd), lambda i: (i, 0)),
            pl.BlockSpec((r, hd), lambda i: (i, 0)),
            pl.BlockSpec((r, hd), lambda i: (i, 0)),
            pl.BlockSpec(wc.shape, lambda i: (0, 0)),
            pl.BlockSpec((1, hd), lambda i: (0, 0)),
        ],
        out_specs=pl.BlockSpec((r, hd), lambda i: (i, 0)),
        out_shape=jax.ShapeDtypeStruct((e_, hd), jnp.float32),
        compiler_params=pltpu.CompilerParams(dimension_semantics=("parallel",)),
    )(gm, h, h0, wc, bc2)


def _tc_final(x, p0, p1, batch3, wa, wb, be2, wf, bf2, g_):
    n_, hd = x.shape
    rn = 1000

    def body(x_ref, p0_ref, p1_ref, b3_ref, wa_ref, wb_ref, be_ref, wf_ref,
             bf_ref, o_ref):
        s = p0_ref[...] + p1_ref[...]
        t = jnp.dot(x_ref[...], wa_ref[...], preferred_element_type=jnp.float32)
        t = t + jnp.dot(s, wb_ref[...], preferred_element_type=jnp.float32)
        t = jnp.maximum(t + be_ref[...], 0.0)
        z = jnp.dot(t, wf_ref[...], preferred_element_type=jnp.float32)
        bvec = b3_ref[0]
        oht = (bvec == lax.broadcasted_iota(jnp.int32, (g_, rn), 0))
        contrib = jnp.dot(oht.astype(jnp.float32), z,
                          preferred_element_type=jnp.float32)
        i = pl.program_id(0)

        @pl.when(i == 0)
        def _():
            o_ref[...] = contrib + bf_ref[...]

        @pl.when(i != 0)
        def _():
            o_ref[...] = o_ref[...] + contrib

    out = pl.pallas_call(
        body,
        grid=(n_ // rn,),
        in_specs=[
            pl.BlockSpec((rn, hd), lambda i: (i, 0)),
            pl.BlockSpec((rn, hd), lambda i: (i, 0)),
            pl.BlockSpec((rn, hd), lambda i: (i, 0)),
            pl.BlockSpec((1, 1, rn), lambda i: (i, 0, 0)),
            pl.BlockSpec(wa.shape, lambda i: (0, 0)),
            pl.BlockSpec(wb.shape, lambda i: (0, 0)),
            pl.BlockSpec((1, hd), lambda i: (0, 0)),
            pl.BlockSpec(wf.shape, lambda i: (0, 0)),
            pl.BlockSpec((1, 1), lambda i: (0, 0)),
        ],
        out_specs=pl.BlockSpec((g_, 1), lambda i: (0, 0)),
        out_shape=jax.ShapeDtypeStruct((g_, 1), jnp.float32),
    )(x, p0, p1, batch3, wa, wb, be2, wf, bf2)
    return out[:, 0]


def kernel(x, edge_index, edge_attr, batch, W_init, b_init, Wc0, bc0, Wc1,
           bc1, Wc2, bc2, W_e2n, b_e2n, W_ffn, b_ffn):
    n_, fn = x.shape
    e_ = edge_attr.shape[0]
    h_ = W_init.shape[1]
    g_ = 64
    eh = (e_ // 2 + NW * CW * 32 - 1) // (NW * CW * 32) * (NW * CW * 32)

    row = edge_index[0]
    dst = edge_index[1]
    idx = {}
    for tag, arr in (("rA", row[:eh]), ("rB", row[eh:]),
                     ("dA", dst[:eh]), ("dB", dst[eh:])):
        idx[tag] = arr.reshape(NW, -1, CW)
    zeros_blk = jnp.zeros((NPAD // NS, h_), jnp.float32)
    x1 = x.reshape(1, n_, fn)
    eat = edge_attr.T
    w1, w2 = W_init[:fn], W_init[fn:]
    b2 = b_init.reshape(1, h_)

    g0a = _sc_gather_staged(x1, idx["rA"])
    g0b = _sc_gather_staged(x1, idx["rB"])
    h0a = _tc_h0(g0a, eat[:, :eh], w1, w2, b2)
    h0b = _tc_h0(g0b, eat[:, eh:], w1, w2, b2)
    ha, hb = h0a, h0b
    for wc, bc in ((Wc0, bc0), (Wc1, bc1), (Wc2, bc2)):
        pa = _sc_scatter(ha, idx["dA"], zeros_blk)
        pb = _sc_scatter(hb, idx["dB"], pa)
        ga = _sc_gather_staged(pb, idx["rA"])
        gb = _sc_gather_staged(pb, idx["rB"])
        ha = _tc_conv(ga, ha, h0a, wc, bc.reshape(1, h_))
        hb = _tc_conv(gb, hb, h0b, wc, bc.reshape(1, h_))
    pa = _sc_scatter(ha, idx["dA"], zeros_blk)
    pb = _sc_scatter(hb, idx["dB"], pa)
    batch3 = batch.reshape(-1, 1, 1000)
    out = _tc_final(x, pb[0], pb[1], batch3, W_e2n[:fn], W_e2n[fn:],
                    b_e2n.reshape(1, h_), W_ffn, b_ffn.reshape(1, 1), g_)
    return out


# speedup vs baseline: 1.0706x; 1.0706x over previous
import functools

import jax
import jax.numpy as jnp
from jax import lax
from jax.experimental import pallas as pl
from jax.experimental.pallas import tpu as pltpu
from jax.experimental.pallas import tpu_sc as plsc

NC = 2
NS = 16
NW = NC * NS
CW = 80
NPAD = 10240
SR = 80


def _sc_mesh():
    return plsc.VectorSubcoreMesh(core_axis_name="c", subcore_axis_name="s")


def _sc_gather_staged(table_bf, idx3):
    npad, hd = table_bf.shape
    nw, ch, cw = idx3.shape
    per_w = ch * cw
    rps = NPAD // NS

    @functools.partial(
        pl.kernel,
        out_type=jax.ShapeDtypeStruct((nw * per_w, hd), jnp.int32),
        mesh=_sc_mesh(),
        scratch_types=[
            pltpu.VMEM((ch, cw), jnp.int32),
            pltpu.VMEM((cw, hd), jnp.int32),
            pltpu.VMEM((cw, hd), jnp.int32),
            pltpu.VMEM_SHARED((NPAD, hd), jnp.int32),
            pltpu.SemaphoreType.DMA,
            pltpu.SemaphoreType.DMA,
            pltpu.SemaphoreType.DMA,
            pltpu.SemaphoreType.DMA,
        ],
    )
    def k(tab, idx, out, idx_v, b0, b1, acc, sg0, sg1, sw0, sw1):
        cid = lax.axis_index("c")
        sid = lax.axis_index("s")
        wid = sid * NC + cid
        base = wid * per_w
        pltpu.sync_copy(idx.at[wid], idx_v)
        pltpu.sync_copy(tab.at[pl.ds(sid * rps, rps)],
                        acc.at[pl.ds(sid * rps, rps)])
        plsc.subcore_barrier()

        pltpu.async_copy(acc.at[idx_v.at[0]], b0, sg0)

        def body(kk, carry):
            j0 = 2 * kk
            c1 = pltpu.async_copy(acc.at[idx_v.at[j0 + 1]], b1, sg1)
            pltpu.make_async_copy(acc.at[idx_v.at[j0]], b0, sg0).wait()
            w0 = pltpu.async_copy(b0, out.at[pl.ds(base + j0 * cw, cw)], sw0)
            c1.wait()
            w1 = pltpu.async_copy(b1, out.at[pl.ds(base + (j0 + 1) * cw, cw)], sw1)
            w0.wait()

            @pl.when(j0 + 2 < ch)
            def _():
                pltpu.async_copy(acc.at[idx_v.at[j0 + 2]], b0, sg0)

            w1.wait()
            return carry

        lax.fori_loop(0, ch // 2, body, 0)
        if ch % 2 == 1:
            jl = ch - 1
            pltpu.make_async_copy(acc.at[idx_v.at[jl]], b0, sg0).wait()
            pltpu.sync_copy(b0, out.at[pl.ds(base + jl * cw, cw)])

    return k(table_bf, idx3)


def _tc_cast_table(planes):
    p_, nr, hd = planes.shape
    r = 2048 if nr % 2048 == 0 else 2000

    def body(p_ref, o_ref):
        s = p_ref[0]
        if p_ == 2:
            s = s + p_ref[1]
        u = lax.bitcast_convert_type(s, jnp.int32) + 0x8000
        lo = lax.shift_right_logical(u[:, :hd // 2], 16)
        hi = jnp.bitwise_and(u[:, hd // 2:], jnp.int32(-65536))
        o_ref[...] = jnp.bitwise_or(lo, hi)

    return pl.pallas_call(
        body,
        grid=(nr // r,),
        in_specs=[pl.BlockSpec((p_, r, hd), lambda i: (0, i, 0))],
        out_specs=pl.BlockSpec((r, hd // 2), lambda i: (i, 0)),
        out_shape=jax.ShapeDtypeStruct((NPAD, hd // 2), jnp.int32),
        compiler_params=pltpu.CompilerParams(dimension_semantics=("arbitrary",)),
    )(planes)


def _sc_scatter(hmat, idx3, init):
    nw, ch, cw = idx3.shape
    hd = hmat.shape[1]
    per_w = ch * cw
    rps = NPAD // NS
    chained = init.ndim == 3

    @functools.partial(
        pl.kernel,
        out_type=jax.ShapeDtypeStruct((NC, NPAD, hd), jnp.float32),
        mesh=_sc_mesh(),
        scratch_types=[
            pltpu.VMEM((ch, cw), jnp.int32),
            pltpu.VMEM((cw, hd), jnp.float32),
            pltpu.VMEM((cw, hd), jnp.float32),
            pltpu.VMEM_SHARED((NPAD, hd), jnp.float32),
            pltpu.SemaphoreType.DMA,
            pltpu.SemaphoreType.DMA,
            pltpu.SemaphoreType.DMA,
            pltpu.SemaphoreType.DMA,
        ],
    )
    def k(h, idx, zb, out, idx_v, b0, b1, acc, sl0, sl1, ss0, ss1):
        cid = lax.axis_index("c")
        sid = lax.axis_index("s")
        wid = sid * NC + cid
        base = wid * per_w
        pltpu.sync_copy(idx.at[wid], idx_v)
        if chained:
            pltpu.sync_copy(zb.at[cid, pl.ds(sid * rps, rps)],
                            acc.at[pl.ds(sid * rps, rps)])
        else:
            pltpu.sync_copy(zb, acc.at[pl.ds(sid * rps, rps)])
        plsc.subcore_barrier()
        pltpu.async_copy(h.at[pl.ds(base, cw)], b0, sl0)

        def body(kk, carry):
            j0 = 2 * kk
            c1 = pltpu.async_copy(h.at[pl.ds(base + (j0 + 1) * cw, cw)], b1, sl1)
            pltpu.make_async_copy(h.at[pl.ds(base, cw)], b0, sl0).wait()
            s0 = pltpu.async_copy(b0, acc.at[idx_v.at[j0]], ss0, add=True)
            c1.wait()
            s1 = pltpu.async_copy(b1, acc.at[idx_v.at[j0 + 1]], ss1, add=True)
            s0.wait()

            @pl.when(j0 + 2 < ch)
            def _():
                pltpu.async_copy(h.at[pl.ds(base + (j0 + 2) * cw, cw)], b0, sl0)

            s1.wait()

            @pl.when(j0 + 3 < ch)
            def _():
                pltpu.async_copy(h.at[pl.ds(base + (j0 + 3) * cw, cw)], b1, sl1)

            return carry

        lax.fori_loop(0, ch // 2, body, 0)
        if ch % 2 == 1:
            jl = ch - 1
            pltpu.make_async_copy(h.at[pl.ds(base, cw)], b0, sl0).wait()
            pltpu.sync_copy(b0, acc.at[idx_v.at[jl]], add=True)
        plsc.subcore_barrier()
        pltpu.sync_copy(acc.at[pl.ds(sid * rps, rps)],
                        out.at[cid, pl.ds(sid * rps, rps)])

    return k(hmat, idx3, init)


def _tc_h0(g0, eat, w1, w2, b2):
    e_, hd = g0.shape[0], w1.shape[1]
    r = 2560
    fe = eat.shape[0]

    def body(g_ref, e_ref, w1_ref, w2_ref, b_ref, o_ref):
        gi = g_ref[...]
        g32 = jnp.concatenate(
            [lax.bitcast_convert_type(lax.shift_left(gi, 16), jnp.float32),
             lax.bitcast_convert_type(
                 jnp.bitwise_and(gi, jnp.int32(-65536)), jnp.float32)],
            axis=1)
        acc = jnp.dot(g32, w1_ref[...], preferred_element_type=jnp.float32)
        acc = acc + lax.dot_general(
            e_ref[...], w2_ref[...], (((0,), (0,)), ((), ())),
            preferred_element_type=jnp.float32)
        o_ref[...] = jnp.maximum(acc + b_ref[...], 0.0)

    return pl.pallas_call(
        body,
        grid=(e_ // r,),
        in_specs=[
            pl.BlockSpec((r, g0.shape[1]), lambda i: (i, 0)),
            pl.BlockSpec((fe, r), lambda i: (0, i)),
            pl.BlockSpec(w1.shape, lambda i: (0, 0)),
            pl.BlockSpec(w2.shape, lambda i: (0, 0)),
            pl.BlockSpec((1, hd), lambda i: (0, 0)),
        ],
        out_specs=pl.BlockSpec((r, hd), lambda i: (i, 0)),
        out_shape=jax.ShapeDtypeStruct((e_, hd), jnp.float32),
        compiler_params=pltpu.CompilerParams(dimension_semantics=("parallel",)),
    )(g0, eat, w1, w2, b2)


def _tc_conv(gm, h, h0, wc, bc2):
    e_, hd = h.shape
    r = 2560

    def body(g_ref, h_ref, h0_ref, w_ref, b_ref, o_ref):
        hb = h_ref[...]
        up = jnp.concatenate([hb[1:], hb[:1]], axis=0)
        dn = jnp.concatenate([hb[-1:], hb[:-1]], axis=0)
        even = (lax.broadcasted_iota(jnp.int32, (r, 1), 0) % 2) == 0
        rev = jnp.where(even, up, dn)
        gi = g_ref[...]
        g32 = jnp.concatenate(
            [lax.bitcast_convert_type(lax.shift_left(gi, 16), jnp.float32),
             lax.bitcast_convert_type(
                 jnp.bitwise_and(gi, jnp.int32(-65536)), jnp.float32)],
            axis=1)
        d = g32 - rev
        acc = jnp.dot(d, w_ref[...], preferred_element_type=jnp.float32)
        o_ref[...] = jnp.maximum(acc + b_ref[...] + h0_ref[...], 0.0)

    return pl.pallas_call(
        body,
        grid=(e_ // r,),
        in_specs=[
            pl.BlockSpec((r, hd // 2), lambda i: (i, 0)),
            pl.BlockSpec((r, hd), lambda i: (i, 0)),
            pl.BlockSpec((r, hd), lambda i: (i, 0)),
            pl.BlockSpec(wc.shape, lambda i: (0, 0)),
            pl.BlockSpec((1, hd), lambda i: (0, 0)),
        ],
        out_specs=pl.BlockSpec((r, hd), lambda i: (i, 0)),
        out_shape=jax.ShapeDtypeStruct((e_, hd), jnp.float32),
        compiler_params=pltpu.CompilerParams(dimension_semantics=("parallel",)),
    )(gm, h, h0, wc, bc2)


def _tc_final(x, p0, p1, batch3, wa, wb, be2, wf, bf2, g_):
    n_, hd = x.shape
    rn = 1000

    def body(x_ref, p0_ref, p1_ref, b3_ref, wa_ref, wb_ref, be_ref, wf_ref,
             bf_ref, o_ref):
        s = p0_ref[...] + p1_ref[...]
        t = jnp.dot(x_ref[...], wa_ref[...], preferred_element_type=jnp.float32)
        t = t + jnp.dot(s, wb_ref[...], preferred_element_type=jnp.float32)
        t = jnp.maximum(t + be_ref[...], 0.0)
        z = jnp.dot(t, wf_ref[...], preferred_element_type=jnp.float32)
        bvec = b3_ref[0]
        oht = (bvec == lax.broadcasted_iota(jnp.int32, (g_, rn), 0))
        contrib = jnp.dot(oht.astype(jnp.float32), z,
                          preferred_element_type=jnp.float32)
        i = pl.program_id(0)

        @pl.when(i == 0)
        def _():
            o_ref[...] = contrib + bf_ref[...]

        @pl.when(i != 0)
        def _():
            o_ref[...] = o_ref[...] + contrib

    out = pl.pallas_call(
        body,
        grid=(n_ // rn,),
        in_specs=[
            pl.BlockSpec((rn, hd), lambda i: (i, 0)),
            pl.BlockSpec((rn, hd), lambda i: (i, 0)),
            pl.BlockSpec((rn, hd), lambda i: (i, 0)),
            pl.BlockSpec((1, 1, rn), lambda i: (i, 0, 0)),
            pl.BlockSpec(wa.shape, lambda i: (0, 0)),
            pl.BlockSpec(wb.shape, lambda i: (0, 0)),
            pl.BlockSpec((1, hd), lambda i: (0, 0)),
            pl.BlockSpec(wf.shape, lambda i: (0, 0)),
            pl.BlockSpec((1, 1), lambda i: (0, 0)),
        ],
        out_specs=pl.BlockSpec((g_, 1), lambda i: (0, 0)),
        out_shape=jax.ShapeDtypeStruct((g_, 1), jnp.float32),
    )(x, p0, p1, batch3, wa, wb, be2, wf, bf2)
    return out[:, 0]


def kernel(x, edge_index, edge_attr, batch, W_init, b_init, Wc0, bc0, Wc1,
           bc1, Wc2, bc2, W_e2n, b_e2n, W_ffn, b_ffn):
    n_, fn = x.shape
    e_ = edge_attr.shape[0]
    h_ = W_init.shape[1]
    g_ = 64
    eh = (e_ // 2 + NW * CW * 32 - 1) // (NW * CW * 32) * (NW * CW * 32)

    row = edge_index[0]
    dst = edge_index[1]
    idx = {}
    for tag, arr in (("rA", row[:eh]), ("rB", row[eh:]),
                     ("dA", dst[:eh]), ("dB", dst[eh:])):
        idx[tag] = arr.reshape(NW, -1, CW)
    zeros_blk = jnp.zeros((NPAD // NS, h_), jnp.float32)
    x1 = x.reshape(1, n_, fn)
    eat = edge_attr.T
    w1, w2 = W_init[:fn], W_init[fn:]
    b2 = b_init.reshape(1, h_)

    xbf = _tc_cast_table(x1)
    g0a = _sc_gather_staged(xbf, idx["rA"])
    g0b = _sc_gather_staged(xbf, idx["rB"])
    h0a = _tc_h0(g0a, eat[:, :eh], w1, w2, b2)
    h0b = _tc_h0(g0b, eat[:, eh:], w1, w2, b2)
    ha, hb = h0a, h0b
    for wc, bc in ((Wc0, bc0), (Wc1, bc1), (Wc2, bc2)):
        pa = _sc_scatter(ha, idx["dA"], zeros_blk)
        pb = _sc_scatter(hb, idx["dB"], pa)
        tb = _tc_cast_table(pb)
        ga = _sc_gather_staged(tb, idx["rA"])
        gb = _sc_gather_staged(tb, idx["rB"])
        ha = _tc_conv(ga, ha, h0a, wc, bc.reshape(1, h_))
        hb = _tc_conv(gb, hb, h0b, wc, bc.reshape(1, h_))
    pa = _sc_scatter(ha, idx["dA"], zeros_blk)
    pb = _sc_scatter(hb, idx["dB"], pa)
    batch3 = batch.reshape(-1, 1, 1000)
    out = _tc_final(x, pb[0], pb[1], batch3, W_e2n[:fn], W_e2n[fn:],
                    b_e2n.reshape(1, h_), W_ffn, b_ffn.reshape(1, 1), g_)
    return out
